# baseline (speedup 1.0000x reference)
import jax
import jax.numpy as jnp
from jax import lax
from jax.experimental import pallas as pl

OMEGA_K = 256
NBITS = 31
BBLK = 4


def _thr_body(x_ref, o_ref):
    bits = lax.bitcast_convert_type(jnp.abs(x_ref[...]), jnp.int32)
    w = bits.shape[1]
    ones = jnp.ones((BBLK, 8, w), jnp.bfloat16)
    dn = (((2,), (1,)), ((0,), (0,)))

    def step(i, t):
        cand = t | (1 << (30 - i))
        maskf = (bits >= cand[:, None, :]).astype(jnp.bfloat16)
        cnt = lax.dot_general(ones, maskf, dn,
                              preferred_element_type=jnp.float32)[:, 0, :]
        return jnp.where(cnt >= float(OMEGA_K), cand, t)

    t0 = jnp.zeros((BBLK, 128), jnp.int32)
    thr = lax.fori_loop(0, NBITS, step, t0)
    o_ref[...] = jnp.where(bits >= thr[:, None, :], x_ref[...], 0.0)


def kernel(x):
    b, w, d = x.shape
    return pl.pallas_call(
        _thr_body,
        grid=(b // BBLK,),
        in_specs=[pl.BlockSpec((BBLK, w, d), lambda i: (i, 0, 0))],
        out_specs=pl.BlockSpec((BBLK, w, d), lambda i: (i, 0, 0)),
        out_shape=jax.ShapeDtypeStruct(x.shape, x.dtype),
    )(x)
